# baseline (speedup 1.0000x reference)
import jax
import jax.numpy as jnp
from jax import lax
from jax.experimental import pallas as pl
from jax.experimental.pallas import tpu as pltpu
from jax.experimental.pallas import tpu_sc as plsc

EMBED = 64
NC = 2
NS = 16
NW = NC * NS
L = 16


def _body(sg_idx_hbm, wy_idx_hbm, mu_idx_hbm, sg_hbm, wy_hbm, mu_hbm,
          out_hbm, sgi_v, wyi_v, mui_v, sg_v, wy_v, mu_v, out_v, sem):
    bpw = sgi_v.shape[0]
    wid = lax.axis_index("s") * NC + lax.axis_index("c")
    base = wid * bpw
    cps = [
        pltpu.async_copy(sg_idx_hbm.at[pl.ds(base, bpw)], sgi_v, sem),
        pltpu.async_copy(wy_idx_hbm.at[pl.ds(base, bpw)], wyi_v, sem),
        pltpu.async_copy(mu_idx_hbm.at[pl.ds(base, bpw)], mui_v, sem),
        pltpu.async_copy(sg_hbm, sg_v, sem),
        pltpu.async_copy(wy_hbm, wy_v, sem),
        pltpu.async_copy(mu_hbm, mu_v, sem),
    ]
    for cp in cps:
        cp.wait()

    lanes = lax.iota(jnp.int32, L)

    def group(g, carry):
        off = g * L
        sgi = sgi_v[pl.ds(off, L)] * EMBED
        wyi = wyi_v[pl.ds(off, L)] * EMBED
        mui = mui_v[pl.ds(off, L)] * EMBED
        row = (lanes + off) * EMBED
        for d in range(EMBED):
            dvec = (lanes + d) & (EMBED - 1)
            r = (plsc.load_gather(sg_v, [sgi + dvec])
                 + plsc.load_gather(wy_v, [wyi + dvec])
                 + plsc.load_gather(mu_v, [mui + dvec]))
            plsc.store_scatter(out_v, [row + dvec], r)
        return carry

    half = bpw // L // 2
    lax.fori_loop(0, half, group, 0)
    first = pltpu.async_copy(
        out_v.at[pl.ds(0, half * L * EMBED)],
        out_hbm.at[pl.ds(base * EMBED, half * L * EMBED)], sem)
    lax.fori_loop(half, bpw // L, group, 0)
    pltpu.sync_copy(
        out_v.at[pl.ds(half * L * EMBED, half * L * EMBED)],
        out_hbm.at[pl.ds((base + half * L) * EMBED, half * L * EMBED)])
    first.wait()


def kernel(space_group, wyckoff_letter, multiplicity, sg_table,
           wyckoff_table, mult_table):
    B = space_group.shape[0]
    bpw = B // NW
    sg = space_group.astype(jnp.int32)
    wy = wyckoff_letter.astype(jnp.int32)
    mu = multiplicity.astype(jnp.int32)
    mesh = plsc.VectorSubcoreMesh(core_axis_name="c", subcore_axis_name="s")
    run = pl.kernel(
        _body,
        mesh=mesh,
        compiler_params=pltpu.CompilerParams(needs_layout_passes=False),
        out_type=jax.ShapeDtypeStruct((B * EMBED,), jnp.float32),
        scratch_types=[
            pltpu.VMEM((bpw,), jnp.int32),
            pltpu.VMEM((bpw,), jnp.int32),
            pltpu.VMEM((bpw,), jnp.int32),
            pltpu.VMEM((sg_table.size,), jnp.float32),
            pltpu.VMEM((wyckoff_table.size,), jnp.float32),
            pltpu.VMEM((mult_table.size,), jnp.float32),
            pltpu.VMEM((bpw * EMBED,), jnp.float32),
            pltpu.SemaphoreType.DMA,
        ],
    )
    out = run(sg, wy, mu, sg_table.reshape(-1), wyckoff_table.reshape(-1),
              mult_table.reshape(-1))
    return out.reshape(B, EMBED)


# speedup vs baseline: 1.0019x; 1.0019x over previous
import jax
import jax.numpy as jnp
from jax import lax
from jax.experimental import pallas as pl
from jax.experimental.pallas import tpu as pltpu
from jax.experimental.pallas import tpu_sc as plsc

EMBED = 64
NC = 2
NS = 16
NW = NC * NS
L = 16


def _body(sg_idx_hbm, wy_idx_hbm, mu_idx_hbm, sg_hbm, wy_hbm, mu_hbm,
          out_hbm, sgi_v, wyi_v, mui_v, sg_v, wy_v, mu_v, out_v, sem):
    bpw = sgi_v.shape[0]
    wid = lax.axis_index("s") * NC + lax.axis_index("c")
    base = wid * bpw
    cps = [
        pltpu.async_copy(sg_idx_hbm.at[pl.ds(base, bpw)], sgi_v, sem),
        pltpu.async_copy(wy_idx_hbm.at[pl.ds(base, bpw)], wyi_v, sem),
        pltpu.async_copy(mu_idx_hbm.at[pl.ds(base, bpw)], mui_v, sem),
        pltpu.async_copy(sg_hbm, sg_v, sem),
        pltpu.async_copy(wy_hbm, wy_v, sem),
        pltpu.async_copy(mu_hbm, mu_v, sem),
    ]
    for cp in cps:
        cp.wait()

    lanes = lax.iota(jnp.int32, L)

    def group(g, carry):
        off = g * L
        sgi = sgi_v[pl.ds(off, L)] * EMBED
        wyi = wyi_v[pl.ds(off, L)] * EMBED
        mui = mui_v[pl.ds(off, L)] * EMBED
        row = (lanes + off) * EMBED
        for d in range(EMBED):
            dvec = (lanes + d) & (EMBED - 1)
            r = (plsc.load_gather(sg_v, [sgi + dvec])
                 + plsc.load_gather(wy_v, [wyi + dvec])
                 + plsc.load_gather(mu_v, [mui + dvec]))
            plsc.store_scatter(out_v, [row + dvec], r)
        return carry

    lax.fori_loop(0, bpw // L, group, 0)
    pltpu.sync_copy(out_v, out_hbm.at[pl.ds(base * EMBED, bpw * EMBED)])


def kernel(space_group, wyckoff_letter, multiplicity, sg_table,
           wyckoff_table, mult_table):
    B = space_group.shape[0]
    bpw = B // NW
    sg = space_group.astype(jnp.int32)
    wy = wyckoff_letter.astype(jnp.int32)
    mu = multiplicity.astype(jnp.int32)
    mesh = plsc.VectorSubcoreMesh(core_axis_name="c", subcore_axis_name="s")
    run = pl.kernel(
        _body,
        mesh=mesh,
        compiler_params=pltpu.CompilerParams(needs_layout_passes=False),
        out_type=jax.ShapeDtypeStruct((B * EMBED,), jnp.float32),
        scratch_types=[
            pltpu.VMEM((bpw,), jnp.int32),
            pltpu.VMEM((bpw,), jnp.int32),
            pltpu.VMEM((bpw,), jnp.int32),
            pltpu.VMEM((sg_table.size,), jnp.float32),
            pltpu.VMEM((wyckoff_table.size,), jnp.float32),
            pltpu.VMEM((mult_table.size,), jnp.float32),
            pltpu.VMEM((bpw * EMBED,), jnp.float32),
            pltpu.SemaphoreType.DMA,
        ],
    )
    out = run(sg, wy, mu, sg_table.reshape(-1), wyckoff_table.reshape(-1),
              mult_table.reshape(-1))
    return out.reshape(B, EMBED)
